# baseline (speedup 1.0000x reference)
import functools

import jax
import jax.numpy as jnp
from jax import lax
from jax.experimental import pallas as pl
from jax.experimental.pallas import tpu as pltpu
from jax.experimental.pallas import tpu_sc as plsc

S = 8192
D = 1024
NC = 2
ROWS = S // NC
CH = 256
NB = 4
AHEAD = 2
NCHUNK = ROWS // CH

_mesh = plsc.ScalarSubcoreMesh(axis_name="c", num_cores=NC)


@functools.partial(
    pl.kernel,
    mesh=_mesh,
    out_type=jax.ShapeDtypeStruct((S, D), jnp.float32),
    scratch_types=(
        [pltpu.MemorySpace.VMEM_SHARED((CH, D), jnp.float32) for _ in range(NB)]
        + [pltpu.SemaphoreType.DMA for _ in range(2 * NB)]
    ),
)
def _copy_table(table_hbm, out_hbm, *scratch):
    bufs = scratch[:NB]
    sin = scratch[NB:2 * NB]
    sout = scratch[2 * NB:]
    cid = lax.axis_index("c")
    base = cid * ROWS

    def start_in(g):
        return pltpu.async_copy(
            table_hbm.at[pl.ds(base + g * CH, CH)], bufs[g % NB], sin[g % NB]
        )

    def start_out(g):
        return pltpu.async_copy(
            bufs[g % NB], out_hbm.at[pl.ds(base + g * CH, CH)], sout[g % NB]
        )

    cin = [None] * NCHUNK
    cout = [None] * NCHUNK
    for g in range(min(AHEAD, NCHUNK)):
        cin[g] = start_in(g)
    waited = set()
    for g in range(NCHUNK):
        cin[g].wait()
        cout[g] = start_out(g)
        n = g + AHEAD
        if n < NCHUNK:
            if n - NB >= 0:
                cout[n - NB].wait()
                waited.add(n - NB)
            cin[n] = start_in(n)
    for g in range(NCHUNK):
        if g not in waited:
            cout[g].wait()


def kernel(inputs, table):
    del inputs
    return _copy_table(table)[None]


# speedup vs baseline: 1.0662x; 1.0662x over previous
import functools

import jax
import jax.numpy as jnp
from jax import lax
from jax.experimental import pallas as pl
from jax.experimental.pallas import tpu as pltpu
from jax.experimental.pallas import tpu_sc as plsc

S = 8192
D = 1024
NC = 2
NS = 16
NW = NC * NS
ROWS = S // NW
CH = 16
NB = 6
AHEAD = 6
NCHUNK = ROWS // CH

_mesh = plsc.VectorSubcoreMesh(core_axis_name="c", subcore_axis_name="s")


@functools.partial(
    pl.kernel,
    mesh=_mesh,
    out_type=jax.ShapeDtypeStruct((S, D), jnp.float32),
    scratch_types=(
        [pltpu.VMEM((CH, D), jnp.float32) for _ in range(NB)]
        + [pltpu.SemaphoreType.DMA for _ in range(2 * NB)]
    ),
)
def _copy_table(table_hbm, out_hbm, *scratch):
    bufs = scratch[:NB]
    sin = scratch[NB:2 * NB]
    sout = scratch[2 * NB:]
    wid = lax.axis_index("s") * NC + lax.axis_index("c")
    base = wid * ROWS

    def start_in(g):
        return pltpu.async_copy(
            table_hbm.at[pl.ds(base + g * CH, CH)], bufs[g % NB], sin[g % NB]
        )

    def start_out(g):
        return pltpu.async_copy(
            bufs[g % NB], out_hbm.at[pl.ds(base + g * CH, CH)], sout[g % NB]
        )

    cin = [None] * NCHUNK
    cout = [None] * NCHUNK
    for g in range(min(AHEAD, NCHUNK)):
        cin[g] = start_in(g)
    waited = set()
    for g in range(NCHUNK):
        cin[g].wait()
        cout[g] = start_out(g)
        n = g + AHEAD
        if n < NCHUNK:
            if n - NB >= 0:
                cout[n - NB].wait()
                waited.add(n - NB)
            cin[n] = start_in(n)
    for g in range(NCHUNK):
        if g not in waited:
            cout[g].wait()


def kernel(inputs, table):
    del inputs
    return _copy_table(table)[None]


# speedup vs baseline: 1.0692x; 1.0028x over previous
import functools

import jax
import jax.numpy as jnp
from jax import lax
from jax.experimental import pallas as pl
from jax.experimental.pallas import tpu as pltpu
from jax.experimental.pallas import tpu_sc as plsc

S = 8192
D = 1024
NC = 2
NS = 16
NW = NC * NS
ROWS = S // NW
CH = 32
NB = 3
AHEAD = 3
NCHUNK = ROWS // CH

_mesh = plsc.VectorSubcoreMesh(core_axis_name="c", subcore_axis_name="s")


@functools.partial(
    pl.kernel,
    mesh=_mesh,
    out_type=jax.ShapeDtypeStruct((S, D), jnp.float32),
    scratch_types=(
        [pltpu.VMEM((CH, D), jnp.float32) for _ in range(NB)]
        + [pltpu.SemaphoreType.DMA for _ in range(2 * NB)]
    ),
)
def _copy_table(table_hbm, out_hbm, *scratch):
    bufs = scratch[:NB]
    sin = scratch[NB:2 * NB]
    sout = scratch[2 * NB:]
    wid = lax.axis_index("s") * NC + lax.axis_index("c")
    base = wid * ROWS

    def start_in(g):
        return pltpu.async_copy(
            table_hbm.at[pl.ds(base + g * CH, CH)], bufs[g % NB], sin[g % NB]
        )

    def start_out(g):
        return pltpu.async_copy(
            bufs[g % NB], out_hbm.at[pl.ds(base + g * CH, CH)], sout[g % NB]
        )

    cin = [None] * NCHUNK
    cout = [None] * NCHUNK
    for g in range(min(AHEAD, NCHUNK)):
        cin[g] = start_in(g)
    waited = set()
    for g in range(NCHUNK):
        cin[g].wait()
        cout[g] = start_out(g)
        n = g + AHEAD
        if n < NCHUNK:
            if n - NB >= 0:
                cout[n - NB].wait()
                waited.add(n - NB)
            cin[n] = start_in(n)
    for g in range(NCHUNK):
        if g not in waited:
            cout[g].wait()


def kernel(inputs, table):
    del inputs
    return _copy_table(table)[None]
